# baseline (speedup 1.0000x reference)
import functools

import jax
import jax.numpy as jnp
from jax.experimental import pallas as pl
from jax.experimental.pallas import tpu as pltpu

_B, _D, _E = 8192, 1024, 16
_TOPK = 2
_LOSS_COEF = 0.01
_CDIMS = (((1,), (1,)), ((), ()))


def _fused_kernel(x_ref, w1_ref, b1_ref, wout_ref, bout_ref, coefs_ref,
                  icpt_ref, out_ref, loss_ref, gates_ref, w1b_ref, imp_ref,
                  load_ref, *, n_steps, sub, n_sub):
    i = pl.program_id(0)

    @pl.when(i == 0)
    def _init():
        w1b_ref[...] = w1_ref[...].astype(jnp.bfloat16)
        imp_ref[...] = jnp.zeros_like(imp_ref)
        load_ref[...] = jnp.zeros_like(load_ref)

    coefs_b = coefs_ref[...].astype(jnp.bfloat16)
    wout = wout_ref[...]

    for h in range(n_sub):
        x = x_ref[pl.ds(h * sub, sub), :]
        xb = x.astype(jnp.bfloat16)

        eo_t = jax.lax.dot_general(coefs_b, xb, _CDIMS,
                                   preferred_element_type=jnp.float32)
        eo_t = eo_t + icpt_ref[...]

        g = jax.lax.dot_general(xb, w1b_ref[...], _CDIMS,
                                preferred_element_type=jnp.float32)
        g = jnp.maximum(g + b1_ref[...], 0.0)

        scores_t = jax.lax.dot_general(wout, g, _CDIMS,
                                       preferred_element_type=jnp.float32)
        scores_t = scores_t + bout_ref[...]

        e_idx = jax.lax.broadcasted_iota(jnp.int32, scores_t.shape, 0)
        m1 = jnp.max(scores_t, axis=0, keepdims=True)
        idx1 = jnp.min(jnp.where(scores_t == m1, e_idx, _E), axis=0,
                       keepdims=True)
        sel1 = e_idx == idx1
        rest = jnp.where(sel1, -jnp.inf, scores_t)
        m2 = jnp.max(rest, axis=0, keepdims=True)
        idx2 = jnp.min(jnp.where(rest == m2, e_idx, _E), axis=0,
                       keepdims=True)
        mask = sel1 | (e_idx == idx2)

        mx = jnp.maximum(m1, 0.0)
        em0 = jnp.exp(-mx)
        denom = jnp.exp(m1 - mx) + jnp.exp(m2 - mx) + (_E - _TOPK) * em0
        gates_t = jnp.where(mask, jnp.exp(scores_t - mx), em0) / denom

        gates_ref[pl.ds(h * sub, sub), :] = gates_t.T
        out_ref[0, pl.ds(h * sub, sub)] = jnp.sum(gates_t * eo_t, axis=0)

        imp_ref[...] += gates_t
        load_ref[...] += (gates_t > 0.0).astype(jnp.float32)

    @pl.when(i == n_steps - 1)
    def _finish():
        def cv2(v):
            mean = jnp.sum(v) / _E
            var = jnp.sum((v - mean) ** 2) / (_E - 1)
            return var / (mean * mean + 1e-10)

        imp = jnp.sum(imp_ref[...], axis=1, keepdims=True)
        load = jnp.sum(load_ref[...], axis=1, keepdims=True)
        loss = (cv2(imp) + cv2(load)) * _LOSS_COEF
        loss_ref[...] = loss.reshape(1, 1)


@jax.jit
def kernel(x, W1, b1, Wout, bout, coefs, intercepts):
    BT = 1024
    SUB = 512
    n_steps = _B // BT

    b1r = b1.reshape(1, _D)
    boutr = bout.reshape(_E, 1)
    icptr = intercepts.reshape(_E, 1)

    out2d, loss2d, gates = pl.pallas_call(
        functools.partial(_fused_kernel, n_steps=n_steps, sub=SUB,
                          n_sub=BT // SUB),
        grid=(n_steps,),
        in_specs=[
            pl.BlockSpec((BT, _D), lambda i: (i, 0)),
            pl.BlockSpec((_D, _D), lambda i: (0, 0)),
            pl.BlockSpec((1, _D), lambda i: (0, 0)),
            pl.BlockSpec((_E, _D), lambda i: (0, 0)),
            pl.BlockSpec((_E, 1), lambda i: (0, 0)),
            pl.BlockSpec((_E, _D), lambda i: (0, 0)),
            pl.BlockSpec((_E, 1), lambda i: (0, 0)),
        ],
        out_specs=[
            pl.BlockSpec((1, BT), lambda i: (0, i)),
            pl.BlockSpec((1, 1), lambda i: (0, 0)),
            pl.BlockSpec((BT, _E), lambda i: (i, 0)),
        ],
        out_shape=[
            jax.ShapeDtypeStruct((1, _B), jnp.float32),
            jax.ShapeDtypeStruct((1, 1), jnp.float32),
            jax.ShapeDtypeStruct((_B, _E), jnp.float32),
        ],
        scratch_shapes=[
            pltpu.VMEM((_D, _D), jnp.bfloat16),
            pltpu.VMEM((_E, SUB), jnp.float32),
            pltpu.VMEM((_E, SUB), jnp.float32),
        ],
        compiler_params=pltpu.CompilerParams(
            dimension_semantics=("arbitrary",),
        ),
    )(x, W1, b1r, Wout, boutr, coefs, icptr)

    return out2d.reshape(_B), loss2d[0, 0], gates


# speedup vs baseline: 1.0779x; 1.0779x over previous
import functools

import jax
import jax.numpy as jnp
from jax.experimental import pallas as pl
from jax.experimental.pallas import tpu as pltpu

_B, _D, _E = 8192, 1024, 16
_TOPK = 2
_LOSS_COEF = 0.01
_CDIMS = (((1,), (1,)), ((), ()))


def _fused_kernel(x_ref, w1_ref, b1_ref, wout_ref, bout_ref, coefs_ref,
                  icpt_ref, out_ref, loss_ref, gates_ref, w1b_ref, imp_ref,
                  load_ref, *, n_steps, sub, n_sub):
    i = pl.program_id(0)

    @pl.when(i == 0)
    def _init():
        w1b_ref[...] = w1_ref[...].astype(jnp.bfloat16)
        imp_ref[...] = jnp.zeros_like(imp_ref)
        load_ref[...] = jnp.zeros_like(load_ref)

    coefs_b = coefs_ref[...].astype(jnp.bfloat16)
    wout = wout_ref[...]
    bout_c = bout_ref[...].reshape(_E, 1)
    icpt_c = icpt_ref[...].reshape(_E, 1)

    for h in range(n_sub):
        x = x_ref[pl.ds(h * sub, sub), :]
        xb = x.astype(jnp.bfloat16)

        eo_t = jax.lax.dot_general(coefs_b, xb, _CDIMS,
                                   preferred_element_type=jnp.float32)
        eo_t = eo_t + icpt_c

        g = jax.lax.dot_general(xb, w1b_ref[...], _CDIMS,
                                preferred_element_type=jnp.float32)
        g = jnp.maximum(g + b1_ref[...], 0.0)

        scores_t = jax.lax.dot_general(wout, g, _CDIMS,
                                       preferred_element_type=jnp.float32)
        scores_t = scores_t + bout_c

        e_idx = jax.lax.broadcasted_iota(jnp.int32, scores_t.shape, 0)
        m1 = jnp.max(scores_t, axis=0, keepdims=True)
        idx1 = jnp.min(jnp.where(scores_t == m1, e_idx, _E), axis=0,
                       keepdims=True)
        sel1 = e_idx == idx1
        rest = jnp.where(sel1, -jnp.inf, scores_t)
        m2 = jnp.max(rest, axis=0, keepdims=True)
        idx2 = jnp.min(jnp.where(rest == m2, e_idx, _E), axis=0,
                       keepdims=True)
        mask = sel1 | (e_idx == idx2)

        mx = jnp.maximum(m1, 0.0)
        em0 = jnp.exp(-mx)
        denom = jnp.exp(m1 - mx) + jnp.exp(m2 - mx) + (_E - _TOPK) * em0
        gates_t = jnp.where(mask, jnp.exp(scores_t - mx), em0) / denom

        gates_ref[pl.ds(h * sub, sub), :] = gates_t.T
        out_ref[pl.ds(h * sub, sub)] = jnp.sum(gates_t * eo_t, axis=0)

        imp_ref[...] += gates_t
        load_ref[...] += (gates_t > 0.0).astype(jnp.float32)

    @pl.when(i == n_steps - 1)
    def _finish():
        def cv2(v):
            mean = jnp.sum(v) / _E
            var = jnp.sum((v - mean) ** 2) / (_E - 1)
            return var / (mean * mean + 1e-10)

        imp = jnp.sum(imp_ref[...], axis=1, keepdims=True)
        load = jnp.sum(load_ref[...], axis=1, keepdims=True)
        loss = (cv2(imp) + cv2(load)) * _LOSS_COEF
        loss_ref[...] = loss.reshape(1, 1)


@jax.jit
def kernel(x, W1, b1, Wout, bout, coefs, intercepts):
    BT = 2048
    SUB = 512
    n_steps = _B // BT

    out2d, loss2d, gates = pl.pallas_call(
        functools.partial(_fused_kernel, n_steps=n_steps, sub=SUB,
                          n_sub=BT // SUB),
        grid=(n_steps,),
        in_specs=[
            pl.BlockSpec((BT, _D), lambda i: (i, 0)),
            pl.BlockSpec((_D, _D), lambda i: (0, 0)),
            pl.BlockSpec((_D,), lambda i: (0,)),
            pl.BlockSpec((_E, _D), lambda i: (0, 0)),
            pl.BlockSpec((_E,), lambda i: (0,)),
            pl.BlockSpec((_E, _D), lambda i: (0, 0)),
            pl.BlockSpec((_E,), lambda i: (0,)),
        ],
        out_specs=[
            pl.BlockSpec((BT,), lambda i: (i,)),
            pl.BlockSpec((1, 1), lambda i: (0, 0)),
            pl.BlockSpec((BT, _E), lambda i: (i, 0)),
        ],
        out_shape=[
            jax.ShapeDtypeStruct((_B,), jnp.float32),
            jax.ShapeDtypeStruct((1, 1), jnp.float32),
            jax.ShapeDtypeStruct((_B, _E), jnp.float32),
        ],
        scratch_shapes=[
            pltpu.VMEM((_D, _D), jnp.bfloat16),
            pltpu.VMEM((_E, SUB), jnp.float32),
            pltpu.VMEM((_E, SUB), jnp.float32),
        ],
        compiler_params=pltpu.CompilerParams(
            dimension_semantics=("arbitrary",),
        ),
    )(x, W1, b1, Wout, bout, coefs, intercepts)

    return out2d, loss2d.reshape(()), gates


# speedup vs baseline: 1.0820x; 1.0038x over previous
import functools

import jax
import jax.numpy as jnp
from jax.experimental import pallas as pl
from jax.experimental.pallas import tpu as pltpu

_B, _D, _E = 8192, 1024, 16
_TOPK = 2
_LOSS_COEF = 0.01
_CDIMS = (((1,), (1,)), ((), ()))


def _fused_kernel(x_ref, w1_ref, b1_ref, wout_ref, bout_ref, coefs_ref,
                  icpt_ref, out_ref, loss_ref, gates_ref, w1b_ref, imp_ref,
                  load_ref, *, n_steps, sub, n_sub):
    i = pl.program_id(0)

    @pl.when(i == 0)
    def _init():
        w1b_ref[...] = w1_ref[...].astype(jnp.bfloat16)
        imp_ref[...] = jnp.zeros_like(imp_ref)
        load_ref[...] = jnp.zeros_like(load_ref)

    coefs_b = coefs_ref[...].astype(jnp.bfloat16)
    wout = wout_ref[...]
    bout_c = bout_ref[...].reshape(_E, 1)
    icpt_c = icpt_ref[...].reshape(_E, 1)

    for h in range(n_sub):
        x = x_ref[pl.ds(h * sub, sub), :]
        xb = x.astype(jnp.bfloat16)

        eo_t = jax.lax.dot_general(coefs_b, xb, _CDIMS,
                                   preferred_element_type=jnp.float32)
        eo_t = eo_t + icpt_c

        g = jax.lax.dot_general(xb, w1b_ref[...], _CDIMS,
                                preferred_element_type=jnp.float32)
        g = jnp.maximum(g + b1_ref[...], 0.0)

        scores_t = jax.lax.dot_general(wout, g, _CDIMS,
                                       preferred_element_type=jnp.float32)
        scores_t = scores_t + bout_c

        e_idx = jax.lax.broadcasted_iota(jnp.int32, scores_t.shape, 0)
        m1 = jnp.max(scores_t, axis=0, keepdims=True)
        idx1 = jnp.min(jnp.where(scores_t == m1, e_idx, _E), axis=0,
                       keepdims=True)
        sel1 = e_idx == idx1
        rest = jnp.where(sel1, -jnp.inf, scores_t)
        m2 = jnp.max(rest, axis=0, keepdims=True)
        idx2 = jnp.min(jnp.where(rest == m2, e_idx, _E), axis=0,
                       keepdims=True)
        mask = sel1 | (e_idx == idx2)

        mx = jnp.maximum(m1, 0.0)
        em0 = jnp.exp(-mx)
        denom = jnp.exp(m1 - mx) + jnp.exp(m2 - mx) + (_E - _TOPK) * em0
        gates_t = jnp.where(mask, jnp.exp(scores_t - mx), em0) / denom

        gates_ref[pl.ds(h * sub, sub), :] = gates_t.T
        out_ref[pl.ds(h * sub, sub)] = jnp.sum(gates_t * eo_t, axis=0)

        imp_ref[...] += gates_t
        load_ref[...] += (gates_t > 0.0).astype(jnp.float32)

    @pl.when(i == n_steps - 1)
    def _finish():
        def cv2(v):
            mean = jnp.sum(v) / _E
            var = jnp.sum((v - mean) ** 2) / (_E - 1)
            return var / (mean * mean + 1e-10)

        imp = jnp.sum(imp_ref[...], axis=1, keepdims=True)
        load = jnp.sum(load_ref[...], axis=1, keepdims=True)
        loss = (cv2(imp) + cv2(load)) * _LOSS_COEF
        loss_ref[...] = loss.reshape(1, 1)


@jax.jit
def kernel(x, W1, b1, Wout, bout, coefs, intercepts):
    BT = 1024
    SUB = 512
    n_steps = _B // BT

    out2d, loss2d, gates = pl.pallas_call(
        functools.partial(_fused_kernel, n_steps=n_steps, sub=SUB,
                          n_sub=BT // SUB),
        grid=(n_steps,),
        in_specs=[
            pl.BlockSpec((BT, _D), lambda i: (i, 0)),
            pl.BlockSpec((_D, _D), lambda i: (0, 0)),
            pl.BlockSpec((_D,), lambda i: (0,)),
            pl.BlockSpec((_E, _D), lambda i: (0, 0)),
            pl.BlockSpec((_E,), lambda i: (0,)),
            pl.BlockSpec((_E, _D), lambda i: (0, 0)),
            pl.BlockSpec((_E,), lambda i: (0,)),
        ],
        out_specs=[
            pl.BlockSpec((BT,), lambda i: (i,)),
            pl.BlockSpec((1, 1), lambda i: (0, 0)),
            pl.BlockSpec((BT, _E), lambda i: (i, 0)),
        ],
        out_shape=[
            jax.ShapeDtypeStruct((_B,), jnp.float32),
            jax.ShapeDtypeStruct((1, 1), jnp.float32),
            jax.ShapeDtypeStruct((_B, _E), jnp.float32),
        ],
        scratch_shapes=[
            pltpu.VMEM((_D, _D), jnp.bfloat16),
            pltpu.VMEM((_E, SUB), jnp.float32),
            pltpu.VMEM((_E, SUB), jnp.float32),
        ],
        compiler_params=pltpu.CompilerParams(
            dimension_semantics=("arbitrary",),
        ),
    )(x, W1, b1, Wout, bout, coefs, intercepts)

    return out2d, loss2d.reshape(()), gates
